# baseline (speedup 1.0000x reference)
import functools

import jax
import jax.numpy as jnp
import numpy as np
from jax import lax
from jax.experimental import pallas as pl
from jax.experimental.pallas import tpu as pltpu
from jax.experimental.pallas import tpu_sc as plsc

N, K, A, D, P = 1024, 32, 4, 256, 64
J = 8
TW = 128
PW = 96
BN = 32
E = BN * K


def _np_consts():
    G = np.zeros((12, 36), np.float32)
    H = np.zeros((9, 36), np.float32)
    Fm = np.zeros((36, 12), np.float32)
    for i in range(3):
        for a_ in range(4):
            for j in range(3):
                G[a_ * 3 + i, i * 12 + a_ * 3 + j] = 1.0
                Fm[i * 12 + a_ * 3 + j, a_ * 3 + j] = 1.0
        for j in range(3):
            for a_ in range(4):
                H[i * 3 + j, i * 12 + a_ * 3 + j] = 1.0
    A48 = np.zeros((12, 48), np.float32)
    B48 = np.zeros((12, 48), np.float32)
    for a_ in range(4):
        for b_ in range(4):
            for c in range(3):
                A48[a_ * 3 + c, a_ * 12 + b_ * 3 + c] = 1.0
                B48[b_ * 3 + c, a_ * 12 + b_ * 3 + c] = 1.0
    S48 = np.zeros((48, 48), np.float32)
    for u in range(48):
        for v in range(48):
            if u // 3 == v // 3:
                S48[u, v] = 1.0
    S12 = S48[:12, :12].copy()
    RepA = np.zeros((8, 64), np.float32)
    TileB = np.zeros((8, 64), np.float32)
    for a_ in range(8):
        for b_ in range(8):
            RepA[a_, a_ * 8 + b_] = 1.0
            TileB[b_, a_ * 8 + b_] = 1.0
    return G, H, Fm, A48, B48, S48, S12, RepA, TileB


_CONSTS = _np_consts()


def _stage1_body(feat, pos12, maskf, ln1s, ln1b,
                 wol, wor, wal, war, wg1, wh1, wl_, wr_, wg2, wh2,
                 a48_c, b48_c, dsc,
                 table_out, pernode_out, wgh1_out, wlr_out, wgh2_out,
                 aw_out, bw_out):
    x = feat[...]
    ones_col = jnp.ones((D, 1), jnp.float32)
    m = jnp.dot(x, ones_col, preferred_element_type=jnp.float32) * (1.0 / D)
    msq = jnp.dot(x * x, ones_col, preferred_element_type=jnp.float32) * (1.0 / D)
    ln = (x - m) * lax.rsqrt(msq - m * m + 1e-5) * ln1s[...] + ln1b[...]
    wnode = jnp.concatenate([wol[...], wor[...], wal[...], war[...]], -1)
    proj = jnp.dot(ln, wnode, preferred_element_type=jnp.float32)
    ol = jax.nn.gelu(proj[:, 0:8])
    orr = proj[:, 8:16]
    al = proj[:, 16:80]
    ar = proj[:, 80:144]

    p = pos12[...]
    nn = p[:, 0:3]
    ca = p[:, 3:6]
    cc = p[:, 6:9]

    def _norm(vv):
        return vv * lax.rsqrt(jnp.sum(vv * vv, -1, keepdims=True) + 1e-8)

    e1 = _norm(cc - ca)
    u = nn - ca
    e2 = _norm(u - jnp.sum(u * e1, -1, keepdims=True) * e1)
    e3 = jnp.concatenate([
        e1[:, 1:2] * e2[:, 2:3] - e1[:, 2:3] * e2[:, 1:2],
        e1[:, 2:3] * e2[:, 0:1] - e1[:, 0:1] * e2[:, 2:3],
        e1[:, 0:1] * e2[:, 1:2] - e1[:, 1:2] * e2[:, 0:1],
    ], -1)
    r9 = jnp.concatenate([
        e1[:, 0:1], e2[:, 0:1], e3[:, 0:1],
        e1[:, 1:2], e2[:, 1:2], e3[:, 1:2],
        e1[:, 2:3], e2[:, 2:3], e3[:, 2:3],
    ], -1)

    mk = maskf[...]
    d = dsc[...]
    dscale = jnp.log1p(jnp.exp(d)) * 0.1
    dcol = jnp.broadcast_to(dscale, (N, 1))
    zpad_t = jnp.zeros((N, TW - 85), jnp.float32)
    zpad_p = jnp.zeros((N, PW - 86), jnp.float32)
    table_out[...] = jnp.concatenate([orr, ar, p[:, 0:12], mk, zpad_t], -1)
    pernode_out[...] = jnp.concatenate([ol, al, r9, ca, mk, dcol, zpad_p], -1)

    wgh2 = jnp.concatenate([wg2[...], wh2[...]], -1)
    wgh1_out[...] = jnp.concatenate([wg1[...], wh1[...]], -1)
    wlr_out[...] = jnp.concatenate([wl_[...], wr_[...]], -1)
    wgh2_out[...] = wgh2
    aw_out[...] = jnp.dot(a48_c[...], wgh2[64:112], preferred_element_type=jnp.float32)
    bw_out[...] = jnp.dot(b48_c[...], wgh2[64:112], preferred_element_type=jnp.float32)


def _stage3_body(pair, gath, pernode, nbr, ln2s, ln2b, ln3s, ln3b,
                 wgh1, wo1, wlr, wgh2, wo2, aw, bw,
                 g_c, h_c, fm_c, a48_c, b48_c, s48_c, s12_c, repa_c, tileb_c,
                 out):
    f32 = jnp.float32

    def mm(a, b):
        return jnp.dot(a, b, preferred_element_type=f32)

    pr = pair[...].reshape(E, P)
    g = gath[...]
    orr_e = g[:, 0:8]
    ar_e = g[:, 8:72]
    pos_e = g[:, 72:84]
    mk_src = g[:, 84:85]

    pn_node = pernode[...]
    per_e = jnp.broadcast_to(pn_node[:, None, :], (BN, K, PW)).reshape(E, PW)
    ol_e = per_e[:, 0:8]
    al_e = per_e[:, 8:72]
    r_e = per_e[:, 72:81]
    t_e = per_e[:, 81:84]
    mk_dst = per_e[:, 84:85]
    dsc_e = per_e[:, 85:86]

    valid = (nbr[...] != -1).astype(f32)
    pmask = mk_dst * mk_src * valid

    ones_col = jnp.ones((P, 1), f32)

    def _ln(x, s, b):
        m = mm(x, ones_col) * (1.0 / P)
        msq = mm(x * x, ones_col) * (1.0 / P)
        return (x - m) * lax.rsqrt(msq - m * m + 1e-5) * s + b

    pn = _ln(pr, ln2s[...], ln2b[...])
    outer = mm(ol_e, repa_c[...]) * mm(orr_e, tileb_c[...])
    additive = al_e + ar_e
    w1 = wgh1[...]
    gh1 = mm(pn, w1[0:64]) + mm(outer, w1[64:128]) + mm(additive, w1[128:192])
    hidden1 = jax.nn.gelu(gh1[:, 0:128]) * gh1[:, 128:256]
    pair2 = pr + mm(hidden1, wo1[...])

    pn2 = _ln(pair2, ln3s[...], ln3b[...])
    t12 = jnp.concatenate([t_e, t_e, t_e, t_e], -1)
    pm = pos_e - t12
    lp = mm(mm(pm, g_c[...]) * mm(r_e, h_c[...]), fm_c[...])
    dirf = lp * lax.rsqrt(mm(lp * lp, s12_c[...]) + 1e-8)
    wl = wlr[...]
    lr = (mm(pn2, wl[0:64]) + dsc_e * mm(lp, wl[64:76])
          + mm(dirf, wl[76:88]))
    left = lr[:, 0:64]
    right3f = lr[:, 64:128].reshape(BN, K, P)[:, 0:J].reshape(BN * J, P)
    lp3f = lp.reshape(BN, K, 12)[:, 0:J].reshape(BN * J, 12)
    pm3 = pmask.reshape(BN, K)[:, 0:J]

    w2 = wgh2[...]
    wpp = w2[0:64]
    base = mm(left, wpp) + dsc_e * mm(lp, aw[...])
    dscn = jnp.broadcast_to(pn_node[:, 85:86][:, None, :],
                            (BN, J, 1)).reshape(BN * J, 1)
    node_terms = (mm(right3f, wpp) - dscn * mm(lp3f, bw[...]))
    lpB = mm(lp3f, b48_c[...])
    lpA = mm(lp, a48_c[...])
    nt3 = node_terms.reshape(BN, J, 128)
    lpB3 = lpB.reshape(BN, J, 48)

    acc_h = jnp.zeros((E, P), f32)
    cnt = jnp.zeros((E, 1), f32)
    for j in range(J):
        ntj = jnp.broadcast_to(nt3[:, j][:, None, :], (BN, K, 128)).reshape(E, 128)
        lpBj = jnp.broadcast_to(lpB3[:, j][:, None, :], (BN, K, 48)).reshape(E, 48)
        pmj = jnp.broadcast_to(pm3[:, j][:, None, None], (BN, K, 1)).reshape(E, 1)
        rel = lpA - lpBj
        dirs = rel * lax.rsqrt(mm(rel * rel, s48_c[...]) + 1e-8)
        gh2 = base + ntj + mm(dirs, w2[112:160])
        ppm = pmask * pmj
        sel = (ppm > 0).astype(f32)
        acc_h = acc_h + jax.nn.gelu(gh2[:, 0:64]) * gh2[:, 64:128] * sel
        cnt = cnt + ppm
    rec = 1.0 / jnp.maximum(cnt, 1.0)
    out[...] = (pair2 + mm(acc_h, wo2[...]) * rec).reshape(BN, K, P)


def _run_stage1(features, pos12, maskf, ln1s, ln1b,
                wol, wor, wal, war, wg1, wh1, wl_, wr_, wg2, wh2,
                a48, b48, dsc):
    full2 = lambda w: pl.BlockSpec(w, lambda: (0, 0))
    f32 = jnp.float32
    return pl.pallas_call(
        _stage1_body,
        grid=(),
        in_specs=[
            full2((N, D)), full2((N, 12)), full2((N, 1)),
            full2((1, D)), full2((1, D)),
            full2((D, 8)), full2((D, 8)), full2((D, P)), full2((D, P)),
            full2((192, 128)), full2((192, 128)),
            full2((88, P)), full2((88, P)),
            full2((160, P)), full2((160, P)),
            full2((12, 48)), full2((12, 48)), full2((1, 1)),
        ],
        out_specs=[full2((N, TW)), full2((N, PW)), full2((192, 256)),
                   full2((88, 128)), full2((160, 128)),
                   full2((12, 128)), full2((12, 128))],
        out_shape=[
            jax.ShapeDtypeStruct((N, TW), f32),
            jax.ShapeDtypeStruct((N, PW), f32),
            jax.ShapeDtypeStruct((192, 256), f32),
            jax.ShapeDtypeStruct((88, 128), f32),
            jax.ShapeDtypeStruct((160, 128), f32),
            jax.ShapeDtypeStruct((12, 128), f32),
            jax.ShapeDtypeStruct((12, 128), f32),
        ],
    )(features, pos12, maskf, ln1s, ln1b,
      wol, wor, wal, war, wg1, wh1, wl_, wr_, wg2, wh2, a48, b48, dsc)


def _run_sc_gather(table, idx_flat):
    info = plsc.get_sparse_core_info()
    nc, ns = info.num_cores, info.num_subcores
    nw = nc * ns
    b = idx_flat.shape[0]
    b_per_w = b // nw
    bc = 256
    nchunk = b_per_w // bc
    mesh = plsc.VectorSubcoreMesh(core_axis_name="c", subcore_axis_name="s")

    @functools.partial(
        pl.kernel, mesh=mesh,
        out_type=jax.ShapeDtypeStruct((b, TW), jnp.float32),
        scratch_types=[
            pltpu.VMEM((bc,), jnp.int32),
            pltpu.VMEM((bc,), jnp.int32),
            pltpu.VMEM((bc, TW), jnp.float32),
            pltpu.VMEM((bc, TW), jnp.float32),
            pltpu.SemaphoreType.DMA,
            pltpu.SemaphoreType.DMA,
            pltpu.SemaphoreType.DMA,
            pltpu.SemaphoreType.DMA,
            pltpu.SemaphoreType.DMA,
            pltpu.SemaphoreType.DMA,
        ],
    )
    def gather_k(table_hbm, idx_hbm, out_hbm,
                 idx_v0, idx_v1, rows_v0, rows_v1,
                 isem0, isem1, gsem0, gsem1, osem0, osem1):
        wid = lax.axis_index("s") * nc + lax.axis_index("c")
        idx_v = [idx_v0, idx_v1]
        rows_v = [rows_v0, rows_v1]
        isem = [isem0, isem1]
        gsem = [gsem0, gsem1]
        osem = [osem0, osem1]
        h_out = [None, None]
        prev = None
        for c in range(nchunk):
            s = c & 1
            base = wid * b_per_w + c * bc
            if prev is not None:
                pg, ps, pbase = prev
                pg.wait()
                h_out[ps] = pltpu.async_copy(
                    rows_v[ps], out_hbm.at[pl.ds(pbase, bc)], osem[ps])
            if h_out[s] is not None:
                h_out[s].wait()
                h_out[s] = None
            pltpu.async_copy(idx_hbm.at[pl.ds(base, bc)], idx_v[s], isem[s]).wait()
            prev = (pltpu.async_copy(table_hbm.at[idx_v[s]], rows_v[s], gsem[s]),
                    s, base)
        pg, ps, pbase = prev
        pg.wait()
        h_out[ps] = pltpu.async_copy(
            rows_v[ps], out_hbm.at[pl.ds(pbase, bc)], osem[ps])
        for s in range(2):
            if h_out[s] is not None:
                h_out[s].wait()

    return gather_k(table, idx_flat)


def _run_stage3(pair, gathered, pernode, nbr, ln2s, ln2b, ln3s, ln3b,
                wgh1, wo1, wlr, wgh2, wo2, aw, bw, consts):
    n_nodes = pair.shape[0]
    grid = (n_nodes // BN,)
    blk = lambda w: pl.BlockSpec(w, lambda i: (0, 0))
    specs = [
        pl.BlockSpec((BN, K, P), lambda i: (i, 0, 0)),
        pl.BlockSpec((E, TW), lambda i: (i, 0)),
        pl.BlockSpec((BN, PW), lambda i: (i, 0)),
        pl.BlockSpec((E, 1), lambda i: (i, 0)),
        blk((1, P)), blk((1, P)), blk((1, P)), blk((1, P)),
        blk((192, 256)), blk((128, P)), blk((88, 128)), blk((160, 128)), blk((P, P)),
        blk((12, 128)), blk((12, 128)),
        blk((12, 36)), blk((9, 36)), blk((36, 12)),
        blk((12, 48)), blk((12, 48)), blk((48, 48)), blk((12, 12)),
        blk((8, 64)), blk((8, 64)),
    ]
    return pl.pallas_call(
        _stage3_body,
        grid=grid,
        in_specs=specs,
        out_specs=pl.BlockSpec((BN, K, P), lambda i: (i, 0, 0)),
        out_shape=jax.ShapeDtypeStruct((n_nodes, K, P), jnp.float32),
    )(pair, gathered, pernode, nbr.reshape(n_nodes * K, 1), ln2s, ln2b, ln3s, ln3b,
      wgh1, wo1, wlr, wgh2, wo2, aw, bw, *consts)


def kernel(features, pair, pos, neighbours, resi, chain, batch, mask,
           ln1_s, ln1_b, ln2_s, ln2_b, ln3_s, ln3_b,
           W_ol, W_or, W_al, W_ar, W_g1, W_h1, W_o1,
           W_l, W_r, W_g2, W_h2, W_o2, d_scale):
    f32 = jnp.float32
    maskf = mask.astype(f32).reshape(N, 1)
    pos12 = pos.astype(f32).reshape(N, 12)
    nbr = neighbours.astype(jnp.int32)
    dsc = jnp.asarray(d_scale, f32).reshape(1, 1)
    consts = tuple(jnp.asarray(c) for c in _CONSTS)
    a48, b48 = consts[3], consts[4]

    table, pernode, wgh1, wlr, wgh2, aw, bw = _run_stage1(
        features.astype(f32), pos12, maskf,
        ln1_s.reshape(1, D), ln1_b.reshape(1, D),
        W_ol, W_or, W_al, W_ar, W_g1, W_h1, W_l, W_r, W_g2, W_h2,
        a48, b48, dsc)

    gathered = _run_sc_gather(table, nbr.reshape(N * K))
    return _run_stage3(
        pair.astype(f32), gathered, pernode, nbr,
        ln2_s.reshape(1, P), ln2_b.reshape(1, P),
        ln3_s.reshape(1, P), ln3_b.reshape(1, P),
        wgh1, W_o1, wlr, wgh2, W_o2, aw, bw, consts)


# speedup vs baseline: 1.0393x; 1.0393x over previous
import functools

import jax
import jax.numpy as jnp
import numpy as np
from jax import lax
from jax.experimental import pallas as pl
from jax.experimental.pallas import tpu as pltpu
from jax.experimental.pallas import tpu_sc as plsc

N, K, A, D, P = 1024, 32, 4, 256, 64
J = 8
TW = 128
PW = 96
BN = 32
E = BN * K


def _np_consts():
    G = np.zeros((12, 36), np.float32)
    H = np.zeros((9, 36), np.float32)
    Fm = np.zeros((36, 12), np.float32)
    for i in range(3):
        for a_ in range(4):
            for j in range(3):
                G[a_ * 3 + i, i * 12 + a_ * 3 + j] = 1.0
                Fm[i * 12 + a_ * 3 + j, a_ * 3 + j] = 1.0
        for j in range(3):
            for a_ in range(4):
                H[i * 3 + j, i * 12 + a_ * 3 + j] = 1.0
    A48 = np.zeros((12, 48), np.float32)
    B48 = np.zeros((12, 48), np.float32)
    for a_ in range(4):
        for b_ in range(4):
            for c in range(3):
                A48[a_ * 3 + c, a_ * 12 + b_ * 3 + c] = 1.0
                B48[b_ * 3 + c, a_ * 12 + b_ * 3 + c] = 1.0
    S48 = np.zeros((48, 48), np.float32)
    for u in range(48):
        for v in range(48):
            if u // 3 == v // 3:
                S48[u, v] = 1.0
    S12 = S48[:12, :12].copy()
    RepA = np.zeros((8, 64), np.float32)
    TileB = np.zeros((8, 64), np.float32)
    for a_ in range(8):
        for b_ in range(8):
            RepA[a_, a_ * 8 + b_] = 1.0
            TileB[b_, a_ * 8 + b_] = 1.0
    return G, H, Fm, A48, B48, S48, S12, RepA, TileB


_CONSTS = _np_consts()


def _stage1_body(feat, pos12, maskf, ln1s, ln1b,
                 wol, wor, wal, war, wg1, wh1, wl_, wr_, wg2, wh2,
                 a48_c, b48_c, dsc,
                 table_out, pernode_out, wgh1_out, wlr_out, wgh2_out,
                 aw_out, bw_out):
    x = feat[...]
    ones_col = jnp.ones((D, 1), jnp.float32)
    m = jnp.dot(x, ones_col, preferred_element_type=jnp.float32) * (1.0 / D)
    msq = jnp.dot(x * x, ones_col, preferred_element_type=jnp.float32) * (1.0 / D)
    ln = (x - m) * lax.rsqrt(msq - m * m + 1e-5) * ln1s[...] + ln1b[...]
    wnode = jnp.concatenate([wol[...], wor[...], wal[...], war[...]], -1)
    proj = jnp.dot(ln, wnode, preferred_element_type=jnp.float32)
    ol = jax.nn.gelu(proj[:, 0:8])
    orr = proj[:, 8:16]
    al = proj[:, 16:80]
    ar = proj[:, 80:144]

    p = pos12[...]
    nn = p[:, 0:3]
    ca = p[:, 3:6]
    cc = p[:, 6:9]

    def _norm(vv):
        return vv * lax.rsqrt(jnp.sum(vv * vv, -1, keepdims=True) + 1e-8)

    e1 = _norm(cc - ca)
    u = nn - ca
    e2 = _norm(u - jnp.sum(u * e1, -1, keepdims=True) * e1)
    e3 = jnp.concatenate([
        e1[:, 1:2] * e2[:, 2:3] - e1[:, 2:3] * e2[:, 1:2],
        e1[:, 2:3] * e2[:, 0:1] - e1[:, 0:1] * e2[:, 2:3],
        e1[:, 0:1] * e2[:, 1:2] - e1[:, 1:2] * e2[:, 0:1],
    ], -1)
    r9 = jnp.concatenate([
        e1[:, 0:1], e2[:, 0:1], e3[:, 0:1],
        e1[:, 1:2], e2[:, 1:2], e3[:, 1:2],
        e1[:, 2:3], e2[:, 2:3], e3[:, 2:3],
    ], -1)

    mk = maskf[...]
    d = dsc[...]
    dscale = jnp.log1p(jnp.exp(d)) * 0.1
    dcol = jnp.broadcast_to(dscale, (N, 1))
    zpad_t = jnp.zeros((N, TW - 85), jnp.float32)
    zpad_p = jnp.zeros((N, PW - 86), jnp.float32)
    table_out[...] = jnp.concatenate([orr, ar, p[:, 0:12], mk, zpad_t], -1)
    pernode_out[...] = jnp.concatenate([ol, al, r9, ca, mk, dcol, zpad_p], -1)

    wgh2 = jnp.concatenate([wg2[...], wh2[...]], -1)
    wgh1_out[...] = jnp.concatenate([wg1[...], wh1[...]], -1)
    wlr_out[...] = jnp.concatenate([wl_[...], wr_[...]], -1)
    wgh2_out[...] = wgh2
    aw_out[...] = jnp.dot(a48_c[...], wgh2[64:112], preferred_element_type=jnp.float32)
    bw_out[...] = jnp.dot(b48_c[...], wgh2[64:112], preferred_element_type=jnp.float32)


def _stage3_body(pair, gath, pernode, nbr, ln2s, ln2b, ln3s, ln3b,
                 wgh1, wo1, wlr, wgh2, wo2, aw, bw,
                 g_c, h_c, fm_c, a48_c, b48_c, s48_c, s12_c, repa_c, tileb_c,
                 out):
    f32 = jnp.float32

    def mm(a, b):
        return jnp.dot(a, b, preferred_element_type=f32)

    pr = pair[...].reshape(E, P)
    g = gath[...]
    orr_e = g[:, 0:8]
    ar_e = g[:, 8:72]
    pos_e = g[:, 72:84]
    mk_src = g[:, 84:85]

    pn_node = pernode[...]
    per_e = jnp.broadcast_to(pn_node[:, None, :], (BN, K, PW)).reshape(E, PW)
    ol_e = per_e[:, 0:8]
    al_e = per_e[:, 8:72]
    r_e = per_e[:, 72:81]
    t_e = per_e[:, 81:84]
    mk_dst = per_e[:, 84:85]
    dsc_e = per_e[:, 85:86]

    valid = (nbr[...] != -1).astype(f32)
    pmask = mk_dst * mk_src * valid

    def _ln(x, s, b):
        m = jnp.mean(x, -1, keepdims=True)
        v = jnp.mean((x - m) ** 2, -1, keepdims=True)
        return (x - m) * lax.rsqrt(v + 1e-5) * s + b

    pn = _ln(pr, ln2s[...], ln2b[...])
    outer = mm(ol_e, repa_c[...]) * mm(orr_e, tileb_c[...])
    additive = al_e + ar_e
    w1 = wgh1[...]
    gh1 = mm(pn, w1[0:64]) + mm(outer, w1[64:128]) + mm(additive, w1[128:192])
    hidden1 = jax.nn.gelu(gh1[:, 0:128]) * gh1[:, 128:256]
    pair2 = pr + mm(hidden1, wo1[...])

    pn2 = _ln(pair2, ln3s[...], ln3b[...])
    t12 = jnp.concatenate([t_e, t_e, t_e, t_e], -1)
    pm = pos_e - t12
    lp = mm(mm(pm, g_c[...]) * mm(r_e, h_c[...]), fm_c[...])
    dirf = lp * lax.rsqrt(mm(lp * lp, s12_c[...]) + 1e-8)
    wl = wlr[...]
    lr = (mm(pn2, wl[0:64]) + dsc_e * mm(lp, wl[64:76])
          + mm(dirf, wl[76:88]))
    left = lr[:, 0:64]
    right3f = lr[:, 64:128].reshape(BN, K, P)[:, 0:J].reshape(BN * J, P)
    lp3f = lp.reshape(BN, K, 12)[:, 0:J].reshape(BN * J, 12)
    pm3 = pmask.reshape(BN, K)[:, 0:J]

    w2 = wgh2[...]
    wpp = w2[0:64]
    base = mm(left, wpp) + dsc_e * mm(lp, aw[...])
    dscn = jnp.broadcast_to(pn_node[:, 85:86][:, None, :],
                            (BN, J, 1)).reshape(BN * J, 1)
    node_terms = (mm(right3f, wpp) - dscn * mm(lp3f, bw[...]))
    lpB = mm(lp3f, b48_c[...])
    lpA = mm(lp, a48_c[...])
    nt3 = node_terms.reshape(BN, J, 128)
    lpB3 = lpB.reshape(BN, J, 48)

    acc_h = jnp.zeros((E, P), f32)
    cnt = jnp.zeros((E, 1), f32)
    for j in range(J):
        ntj = jnp.broadcast_to(nt3[:, j][:, None, :], (BN, K, 128)).reshape(E, 128)
        lpBj = jnp.broadcast_to(lpB3[:, j][:, None, :], (BN, K, 48)).reshape(E, 48)
        pmj = jnp.broadcast_to(pm3[:, j][:, None, None], (BN, K, 1)).reshape(E, 1)
        rel = lpA - lpBj
        dirs = rel * lax.rsqrt(mm(rel * rel, s48_c[...]) + 1e-8)
        gh2 = base + ntj + mm(dirs, w2[112:160])
        ppm = pmask * pmj
        sel = (ppm > 0).astype(f32)
        acc_h = acc_h + jax.nn.gelu(gh2[:, 0:64]) * gh2[:, 64:128] * sel
        cnt = cnt + ppm
    rec = 1.0 / jnp.maximum(cnt, 1.0)
    out[...] = (pair2 + mm(acc_h, wo2[...]) * rec).reshape(BN, K, P)


def _run_stage1(features, pos12, maskf, ln1s, ln1b,
                wol, wor, wal, war, wg1, wh1, wl_, wr_, wg2, wh2,
                a48, b48, dsc):
    full2 = lambda w: pl.BlockSpec(w, lambda: (0, 0))
    f32 = jnp.float32
    return pl.pallas_call(
        _stage1_body,
        grid=(),
        in_specs=[
            full2((N, D)), full2((N, 12)), full2((N, 1)),
            full2((1, D)), full2((1, D)),
            full2((D, 8)), full2((D, 8)), full2((D, P)), full2((D, P)),
            full2((192, 128)), full2((192, 128)),
            full2((88, P)), full2((88, P)),
            full2((160, P)), full2((160, P)),
            full2((12, 48)), full2((12, 48)), full2((1, 1)),
        ],
        out_specs=[full2((N, TW)), full2((N, PW)), full2((192, 256)),
                   full2((88, 128)), full2((160, 128)),
                   full2((12, 128)), full2((12, 128))],
        out_shape=[
            jax.ShapeDtypeStruct((N, TW), f32),
            jax.ShapeDtypeStruct((N, PW), f32),
            jax.ShapeDtypeStruct((192, 256), f32),
            jax.ShapeDtypeStruct((88, 128), f32),
            jax.ShapeDtypeStruct((160, 128), f32),
            jax.ShapeDtypeStruct((12, 128), f32),
            jax.ShapeDtypeStruct((12, 128), f32),
        ],
    )(features, pos12, maskf, ln1s, ln1b,
      wol, wor, wal, war, wg1, wh1, wl_, wr_, wg2, wh2, a48, b48, dsc)


def _run_sc_gather(table, idx_flat):
    info = plsc.get_sparse_core_info()
    nc, ns = info.num_cores, info.num_subcores
    nw = nc * ns
    b = idx_flat.shape[0]
    b_per_w = b // nw
    bc = 256
    nchunk = b_per_w // bc
    mesh = plsc.VectorSubcoreMesh(core_axis_name="c", subcore_axis_name="s")

    @functools.partial(
        pl.kernel, mesh=mesh,
        out_type=jax.ShapeDtypeStruct((b, TW), jnp.float32),
        scratch_types=[
            pltpu.VMEM((bc,), jnp.int32),
            pltpu.VMEM((bc,), jnp.int32),
            pltpu.VMEM((bc, TW), jnp.float32),
            pltpu.VMEM((bc, TW), jnp.float32),
            pltpu.SemaphoreType.DMA,
            pltpu.SemaphoreType.DMA,
            pltpu.SemaphoreType.DMA,
            pltpu.SemaphoreType.DMA,
            pltpu.SemaphoreType.DMA,
            pltpu.SemaphoreType.DMA,
        ],
    )
    def gather_k(table_hbm, idx_hbm, out_hbm,
                 idx_v0, idx_v1, rows_v0, rows_v1,
                 isem0, isem1, gsem0, gsem1, osem0, osem1):
        wid = lax.axis_index("s") * nc + lax.axis_index("c")
        idx_v = [idx_v0, idx_v1]
        rows_v = [rows_v0, rows_v1]
        isem = [isem0, isem1]
        gsem = [gsem0, gsem1]
        osem = [osem0, osem1]
        h_out = [None, None]
        prev = None
        for c in range(nchunk):
            s = c & 1
            base = wid * b_per_w + c * bc
            if prev is not None:
                pg, ps, pbase = prev
                pg.wait()
                h_out[ps] = pltpu.async_copy(
                    rows_v[ps], out_hbm.at[pl.ds(pbase, bc)], osem[ps])
            if h_out[s] is not None:
                h_out[s].wait()
                h_out[s] = None
            pltpu.async_copy(idx_hbm.at[pl.ds(base, bc)], idx_v[s], isem[s]).wait()
            prev = (pltpu.async_copy(table_hbm.at[idx_v[s]], rows_v[s], gsem[s]),
                    s, base)
        pg, ps, pbase = prev
        pg.wait()
        h_out[ps] = pltpu.async_copy(
            rows_v[ps], out_hbm.at[pl.ds(pbase, bc)], osem[ps])
        for s in range(2):
            if h_out[s] is not None:
                h_out[s].wait()

    return gather_k(table, idx_flat)


def _run_stage3(pair, gathered, pernode, nbr, ln2s, ln2b, ln3s, ln3b,
                wgh1, wo1, wlr, wgh2, wo2, aw, bw, consts):
    n_nodes = pair.shape[0]
    grid = (n_nodes // BN,)
    blk = lambda w: pl.BlockSpec(w, lambda i: (0, 0))
    specs = [
        pl.BlockSpec((BN, K, P), lambda i: (i, 0, 0)),
        pl.BlockSpec((E, TW), lambda i: (i, 0)),
        pl.BlockSpec((BN, PW), lambda i: (i, 0)),
        pl.BlockSpec((E, 1), lambda i: (i, 0)),
        blk((1, P)), blk((1, P)), blk((1, P)), blk((1, P)),
        blk((192, 256)), blk((128, P)), blk((88, 128)), blk((160, 128)), blk((P, P)),
        blk((12, 128)), blk((12, 128)),
        blk((12, 36)), blk((9, 36)), blk((36, 12)),
        blk((12, 48)), blk((12, 48)), blk((48, 48)), blk((12, 12)),
        blk((8, 64)), blk((8, 64)),
    ]
    return pl.pallas_call(
        _stage3_body,
        grid=grid,
        in_specs=specs,
        out_specs=pl.BlockSpec((BN, K, P), lambda i: (i, 0, 0)),
        out_shape=jax.ShapeDtypeStruct((n_nodes, K, P), jnp.float32),
    )(pair, gathered, pernode, nbr.reshape(n_nodes * K, 1), ln2s, ln2b, ln3s, ln3b,
      wgh1, wo1, wlr, wgh2, wo2, aw, bw, *consts)


def kernel(features, pair, pos, neighbours, resi, chain, batch, mask,
           ln1_s, ln1_b, ln2_s, ln2_b, ln3_s, ln3_b,
           W_ol, W_or, W_al, W_ar, W_g1, W_h1, W_o1,
           W_l, W_r, W_g2, W_h2, W_o2, d_scale):
    f32 = jnp.float32
    maskf = mask.astype(f32).reshape(N, 1)
    pos12 = pos.astype(f32).reshape(N, 12)
    nbr = neighbours.astype(jnp.int32)
    dsc = jnp.asarray(d_scale, f32).reshape(1, 1)
    consts = tuple(jnp.asarray(c) for c in _CONSTS)
    a48, b48 = consts[3], consts[4]

    table, pernode, wgh1, wlr, wgh2, aw, bw = _run_stage1(
        features.astype(f32), pos12, maskf,
        ln1_s.reshape(1, D), ln1_b.reshape(1, D),
        W_ol, W_or, W_al, W_ar, W_g1, W_h1, W_l, W_r, W_g2, W_h2,
        a48, b48, dsc)

    gathered = _run_sc_gather(table, nbr.reshape(N * K))
    return _run_stage3(
        pair.astype(f32), gathered, pernode, nbr,
        ln2_s.reshape(1, P), ln2_b.reshape(1, P),
        ln3_s.reshape(1, P), ln3_b.reshape(1, P),
        wgh1, W_o1, wlr, wgh2, W_o2, aw, bw, consts)


# speedup vs baseline: 1.0567x; 1.0168x over previous
import functools

import jax
import jax.numpy as jnp
import numpy as np
from jax import lax
from jax.experimental import pallas as pl
from jax.experimental.pallas import tpu as pltpu
from jax.experimental.pallas import tpu_sc as plsc

N, K, A, D, P = 1024, 32, 4, 256, 64
J = 8
TW = 128
PW = 96
BN = 32
E = BN * K


def _np_consts():
    G = np.zeros((12, 36), np.float32)
    H = np.zeros((9, 36), np.float32)
    Fm = np.zeros((36, 12), np.float32)
    for i in range(3):
        for a_ in range(4):
            for j in range(3):
                G[a_ * 3 + i, i * 12 + a_ * 3 + j] = 1.0
                Fm[i * 12 + a_ * 3 + j, a_ * 3 + j] = 1.0
        for j in range(3):
            for a_ in range(4):
                H[i * 3 + j, i * 12 + a_ * 3 + j] = 1.0
    A48 = np.zeros((12, 48), np.float32)
    B48 = np.zeros((12, 48), np.float32)
    for a_ in range(4):
        for b_ in range(4):
            for c in range(3):
                A48[a_ * 3 + c, a_ * 12 + b_ * 3 + c] = 1.0
                B48[b_ * 3 + c, a_ * 12 + b_ * 3 + c] = 1.0
    S48 = np.zeros((48, 48), np.float32)
    for u in range(48):
        for v in range(48):
            if u // 3 == v // 3:
                S48[u, v] = 1.0
    S12 = S48[:12, :12].copy()
    RepA = np.zeros((8, 64), np.float32)
    TileB = np.zeros((8, 64), np.float32)
    for a_ in range(8):
        for b_ in range(8):
            RepA[a_, a_ * 8 + b_] = 1.0
            TileB[b_, a_ * 8 + b_] = 1.0
    return G, H, Fm, A48, B48, S48, S12, RepA, TileB


_CONSTS = _np_consts()


def _stage1_body(feat, pos12, maskf, ln1s, ln1b, wnode, dsc,
                 table_out, pernode_out):
    x = feat[...]
    m = jnp.mean(x, -1, keepdims=True)
    v = jnp.mean((x - m) ** 2, -1, keepdims=True)
    ln = (x - m) * lax.rsqrt(v + 1e-5) * ln1s[...] + ln1b[...]
    proj = jnp.dot(ln, wnode[...], preferred_element_type=jnp.float32)
    ol = jax.nn.gelu(proj[:, 0:8])
    orr = proj[:, 8:16]
    al = proj[:, 16:80]
    ar = proj[:, 80:144]

    p = pos12[...]
    nn = p[:, 0:3]
    ca = p[:, 3:6]
    cc = p[:, 6:9]

    def _norm(vv):
        return vv * lax.rsqrt(jnp.sum(vv * vv, -1, keepdims=True) + 1e-8)

    e1 = _norm(cc - ca)
    u = nn - ca
    e2 = _norm(u - jnp.sum(u * e1, -1, keepdims=True) * e1)
    e3 = jnp.concatenate([
        e1[:, 1:2] * e2[:, 2:3] - e1[:, 2:3] * e2[:, 1:2],
        e1[:, 2:3] * e2[:, 0:1] - e1[:, 0:1] * e2[:, 2:3],
        e1[:, 0:1] * e2[:, 1:2] - e1[:, 1:2] * e2[:, 0:1],
    ], -1)
    r9 = jnp.concatenate([
        e1[:, 0:1], e2[:, 0:1], e3[:, 0:1],
        e1[:, 1:2], e2[:, 1:2], e3[:, 1:2],
        e1[:, 2:3], e2[:, 2:3], e3[:, 2:3],
    ], -1)

    mk = maskf[...]
    d = dsc[...]
    dscale = jnp.log1p(jnp.exp(d)) * 0.1
    dcol = jnp.broadcast_to(dscale, (N, 1))
    zpad_t = jnp.zeros((N, TW - 85), jnp.float32)
    zpad_p = jnp.zeros((N, PW - 86), jnp.float32)
    table_out[...] = jnp.concatenate([orr, ar, p[:, 0:12], mk, zpad_t], -1)
    pernode_out[...] = jnp.concatenate([ol, al, r9, ca, mk, dcol, zpad_p], -1)


def _stage3_body(pair, gath, pernode, nbr, ln2s, ln2b, ln3s, ln3b,
                 wgh1, wo1, wlr, wgh2, wo2, aw, bw,
                 g_c, h_c, fm_c, a48_c, b48_c, s48_c, s12_c, repa_c, tileb_c,
                 out):
    f32 = jnp.float32

    def mm(a, b):
        return jnp.dot(a, b, preferred_element_type=f32)

    pr = pair[...].reshape(E, P)
    g = gath[...]
    orr_e = g[:, 0:8]
    ar_e = g[:, 8:72]
    pos_e = g[:, 72:84]
    mk_src = g[:, 84:85]

    pn_node = pernode[...]
    per_e = jnp.broadcast_to(pn_node[:, None, :], (BN, K, PW)).reshape(E, PW)
    ol_e = per_e[:, 0:8]
    al_e = per_e[:, 8:72]
    r_e = per_e[:, 72:81]
    t_e = per_e[:, 81:84]
    mk_dst = per_e[:, 84:85]
    dsc_e = per_e[:, 85:86]

    valid = (nbr[...] != -1).astype(f32)
    pmask = mk_dst * mk_src * valid

    def _ln(x, s, b):
        m = jnp.mean(x, -1, keepdims=True)
        v = jnp.mean((x - m) ** 2, -1, keepdims=True)
        return (x - m) * lax.rsqrt(v + 1e-5) * s + b

    pn = _ln(pr, ln2s[...], ln2b[...])
    outer = mm(ol_e, repa_c[...]) * mm(orr_e, tileb_c[...])
    additive = al_e + ar_e
    w1 = wgh1[...]
    gh1 = mm(pn, w1[0:64]) + mm(outer, w1[64:128]) + mm(additive, w1[128:192])
    hidden1 = jax.nn.gelu(gh1[:, 0:128]) * gh1[:, 128:256]
    pair2 = pr + mm(hidden1, wo1[...])

    pn2 = _ln(pair2, ln3s[...], ln3b[...])
    t12 = jnp.concatenate([t_e, t_e, t_e, t_e], -1)
    pm = pos_e - t12
    lp = mm(mm(pm, g_c[...]) * mm(r_e, h_c[...]), fm_c[...])
    dirf = lp * lax.rsqrt(mm(lp * lp, s12_c[...]) + 1e-8)
    wl = wlr[...]
    lr = (mm(pn2, wl[0:64]) + dsc_e * mm(lp, wl[64:76])
          + mm(dirf, wl[76:88]))
    left = lr[:, 0:64]
    right3f = lr[:, 64:128].reshape(BN, K, P)[:, 0:J].reshape(BN * J, P)
    lp3f = lp.reshape(BN, K, 12)[:, 0:J].reshape(BN * J, 12)
    pm3 = pmask.reshape(BN, K)[:, 0:J]

    w2 = wgh2[...]
    wpp = w2[0:64]
    base = mm(left, wpp) + dsc_e * mm(lp, aw[...])
    dscn = jnp.broadcast_to(pn_node[:, 85:86][:, None, :],
                            (BN, J, 1)).reshape(BN * J, 1)
    node_terms = (mm(right3f, wpp) - dscn * mm(lp3f, bw[...]))
    lpB = mm(lp3f, b48_c[...])
    lpA = mm(lp, a48_c[...])
    nt3 = node_terms.reshape(BN, J, 128)
    lpB3 = lpB.reshape(BN, J, 48)

    acc_h = jnp.zeros((E, P), f32)
    cnt = jnp.zeros((E, 1), f32)
    for j in range(J):
        ntj = jnp.broadcast_to(nt3[:, j][:, None, :], (BN, K, 128)).reshape(E, 128)
        lpBj = jnp.broadcast_to(lpB3[:, j][:, None, :], (BN, K, 48)).reshape(E, 48)
        pmj = jnp.broadcast_to(pm3[:, j][:, None, None], (BN, K, 1)).reshape(E, 1)
        rel = lpA - lpBj
        dirs = rel * lax.rsqrt(mm(rel * rel, s48_c[...]) + 1e-8)
        gh2 = base + ntj + mm(dirs, w2[112:160])
        ppm = pmask * pmj
        sel = (ppm > 0).astype(f32)
        acc_h = acc_h + jax.nn.gelu(gh2[:, 0:64]) * gh2[:, 64:128] * sel
        cnt = cnt + ppm
    rec = 1.0 / jnp.maximum(cnt, 1.0)
    out[...] = (pair2 + mm(acc_h, wo2[...]) * rec).reshape(BN, K, P)


def _run_stage1(features, pos12, maskf, ln1s, ln1b, wnode, dsc):
    full2 = lambda w: pl.BlockSpec(w, lambda: (0, 0))
    return pl.pallas_call(
        _stage1_body,
        grid=(),
        in_specs=[
            full2((N, D)), full2((N, 12)), full2((N, 1)),
            full2((1, D)), full2((1, D)), full2((D, 144)), full2((1, 1)),
        ],
        out_specs=[full2((N, TW)), full2((N, PW))],
        out_shape=[
            jax.ShapeDtypeStruct((N, TW), jnp.float32),
            jax.ShapeDtypeStruct((N, PW), jnp.float32),
        ],
    )(features, pos12, maskf, ln1s, ln1b, wnode, dsc)


def _run_sc_gather(table, idx_flat):
    info = plsc.get_sparse_core_info()
    nc, ns = info.num_cores, info.num_subcores
    nw = nc * ns
    b = idx_flat.shape[0]
    b_per_w = b // nw
    bc = 256
    nchunk = b_per_w // bc
    mesh = plsc.VectorSubcoreMesh(core_axis_name="c", subcore_axis_name="s")

    @functools.partial(
        pl.kernel, mesh=mesh,
        out_type=jax.ShapeDtypeStruct((b, TW), jnp.float32),
        scratch_types=[
            pltpu.VMEM((bc,), jnp.int32),
            pltpu.VMEM((bc,), jnp.int32),
            pltpu.VMEM((bc, TW), jnp.float32),
            pltpu.VMEM((bc, TW), jnp.float32),
            pltpu.SemaphoreType.DMA,
            pltpu.SemaphoreType.DMA,
            pltpu.SemaphoreType.DMA,
            pltpu.SemaphoreType.DMA,
            pltpu.SemaphoreType.DMA,
            pltpu.SemaphoreType.DMA,
        ],
    )
    def gather_k(table_hbm, idx_hbm, out_hbm,
                 idx_v0, idx_v1, rows_v0, rows_v1,
                 isem0, isem1, gsem0, gsem1, osem0, osem1):
        wid = lax.axis_index("s") * nc + lax.axis_index("c")
        idx_v = [idx_v0, idx_v1]
        rows_v = [rows_v0, rows_v1]
        isem = [isem0, isem1]
        gsem = [gsem0, gsem1]
        osem = [osem0, osem1]
        h_out = [None, None]
        prev = None
        for c in range(nchunk):
            s = c & 1
            base = wid * b_per_w + c * bc
            if prev is not None:
                pg, ps, pbase = prev
                pg.wait()
                h_out[ps] = pltpu.async_copy(
                    rows_v[ps], out_hbm.at[pl.ds(pbase, bc)], osem[ps])
            if h_out[s] is not None:
                h_out[s].wait()
                h_out[s] = None
            pltpu.async_copy(idx_hbm.at[pl.ds(base, bc)], idx_v[s], isem[s]).wait()
            prev = (pltpu.async_copy(table_hbm.at[idx_v[s]], rows_v[s], gsem[s]),
                    s, base)
        pg, ps, pbase = prev
        pg.wait()
        h_out[ps] = pltpu.async_copy(
            rows_v[ps], out_hbm.at[pl.ds(pbase, bc)], osem[ps])
        for s in range(2):
            if h_out[s] is not None:
                h_out[s].wait()

    return gather_k(table, idx_flat)


def _run_stage3(pair, gathered, pernode, nbr, ln2s, ln2b, ln3s, ln3b,
                wgh1, wo1, wlr, wgh2, wo2, aw, bw, consts):
    n_nodes = pair.shape[0]
    grid = (n_nodes // BN,)
    blk = lambda w: pl.BlockSpec(w, lambda i: (0, 0))
    specs = [
        pl.BlockSpec((BN, K, P), lambda i: (i, 0, 0)),
        pl.BlockSpec((E, TW), lambda i: (i, 0)),
        pl.BlockSpec((BN, PW), lambda i: (i, 0)),
        pl.BlockSpec((E, 1), lambda i: (i, 0)),
        blk((1, P)), blk((1, P)), blk((1, P)), blk((1, P)),
        blk((192, 256)), blk((128, P)), blk((88, 128)), blk((160, 128)), blk((P, P)),
        blk((12, 128)), blk((12, 128)),
        blk((12, 36)), blk((9, 36)), blk((36, 12)),
        blk((12, 48)), blk((12, 48)), blk((48, 48)), blk((12, 12)),
        blk((8, 64)), blk((8, 64)),
    ]
    return pl.pallas_call(
        _stage3_body,
        grid=grid,
        in_specs=specs,
        out_specs=pl.BlockSpec((BN, K, P), lambda i: (i, 0, 0)),
        out_shape=jax.ShapeDtypeStruct((n_nodes, K, P), jnp.float32),
    )(pair, gathered, pernode, nbr.reshape(n_nodes * K, 1), ln2s, ln2b, ln3s, ln3b,
      wgh1, wo1, wlr, wgh2, wo2, aw, bw, *consts)


def kernel(features, pair, pos, neighbours, resi, chain, batch, mask,
           ln1_s, ln1_b, ln2_s, ln2_b, ln3_s, ln3_b,
           W_ol, W_or, W_al, W_ar, W_g1, W_h1, W_o1,
           W_l, W_r, W_g2, W_h2, W_o2, d_scale):
    f32 = jnp.float32
    maskf = mask.astype(f32).reshape(N, 1)
    pos12 = pos.astype(f32).reshape(N, 12)
    nbr = neighbours.astype(jnp.int32)
    dsc = jnp.asarray(d_scale, f32).reshape(1, 1)
    consts = tuple(jnp.asarray(c) for c in _CONSTS)
    a48, b48 = consts[3], consts[4]
    wnode = jnp.concatenate([W_ol, W_or, W_al, W_ar], axis=1)
    wgh1 = jnp.concatenate([W_g1, W_h1], axis=1)
    wlr = jnp.concatenate([W_l, W_r], axis=1)
    wgh2 = jnp.concatenate([W_g2, W_h2], axis=1)
    aw = jnp.dot(a48, wgh2[64:112])
    bw = jnp.dot(b48, wgh2[64:112])

    table, pernode = _run_stage1(
        features.astype(f32), pos12, maskf,
        ln1_s.reshape(1, D), ln1_b.reshape(1, D), wnode, dsc)

    gathered = _run_sc_gather(table, nbr.reshape(N * K))
    return _run_stage3(
        pair.astype(f32), gathered, pernode, nbr,
        ln2_s.reshape(1, P), ln2_b.reshape(1, P),
        ln3_s.reshape(1, P), ln3_b.reshape(1, P),
        wgh1, W_o1, wlr, wgh2, W_o2, aw, bw, consts)


# speedup vs baseline: 1.0823x; 1.0242x over previous
import functools

import jax
import jax.numpy as jnp
import numpy as np
from jax import lax
from jax.experimental import pallas as pl
from jax.experimental.pallas import tpu as pltpu
from jax.experimental.pallas import tpu_sc as plsc

N, K, A, D, P = 1024, 32, 4, 256, 64
J = 8
TW = 128
PW = 96
BN = 64
E = BN * K


def _np_consts():
    G = np.zeros((12, 36), np.float32)
    H = np.zeros((9, 36), np.float32)
    Fm = np.zeros((36, 12), np.float32)
    for i in range(3):
        for a_ in range(4):
            for j in range(3):
                G[a_ * 3 + i, i * 12 + a_ * 3 + j] = 1.0
                Fm[i * 12 + a_ * 3 + j, a_ * 3 + j] = 1.0
        for j in range(3):
            for a_ in range(4):
                H[i * 3 + j, i * 12 + a_ * 3 + j] = 1.0
    A48 = np.zeros((12, 48), np.float32)
    B48 = np.zeros((12, 48), np.float32)
    for a_ in range(4):
        for b_ in range(4):
            for c in range(3):
                A48[a_ * 3 + c, a_ * 12 + b_ * 3 + c] = 1.0
                B48[b_ * 3 + c, a_ * 12 + b_ * 3 + c] = 1.0
    S48 = np.zeros((48, 48), np.float32)
    for u in range(48):
        for v in range(48):
            if u // 3 == v // 3:
                S48[u, v] = 1.0
    S12 = S48[:12, :12].copy()
    RepA = np.zeros((8, 64), np.float32)
    TileB = np.zeros((8, 64), np.float32)
    for a_ in range(8):
        for b_ in range(8):
            RepA[a_, a_ * 8 + b_] = 1.0
            TileB[b_, a_ * 8 + b_] = 1.0
    return G, H, Fm, A48, B48, S48, S12, RepA, TileB


_CONSTS = _np_consts()


def _stage1_body(feat, pos12, maskf, ln1s, ln1b, wnode, dsc,
                 table_out, pernode_out):
    x = feat[...]
    m = jnp.mean(x, -1, keepdims=True)
    v = jnp.mean((x - m) ** 2, -1, keepdims=True)
    ln = (x - m) * lax.rsqrt(v + 1e-5) * ln1s[...] + ln1b[...]
    proj = jnp.dot(ln, wnode[...], preferred_element_type=jnp.float32)
    ol = jax.nn.gelu(proj[:, 0:8])
    orr = proj[:, 8:16]
    al = proj[:, 16:80]
    ar = proj[:, 80:144]

    p = pos12[...]
    nn = p[:, 0:3]
    ca = p[:, 3:6]
    cc = p[:, 6:9]

    def _norm(vv):
        return vv * lax.rsqrt(jnp.sum(vv * vv, -1, keepdims=True) + 1e-8)

    e1 = _norm(cc - ca)
    u = nn - ca
    e2 = _norm(u - jnp.sum(u * e1, -1, keepdims=True) * e1)
    e3 = jnp.concatenate([
        e1[:, 1:2] * e2[:, 2:3] - e1[:, 2:3] * e2[:, 1:2],
        e1[:, 2:3] * e2[:, 0:1] - e1[:, 0:1] * e2[:, 2:3],
        e1[:, 0:1] * e2[:, 1:2] - e1[:, 1:2] * e2[:, 0:1],
    ], -1)
    r9 = jnp.concatenate([
        e1[:, 0:1], e2[:, 0:1], e3[:, 0:1],
        e1[:, 1:2], e2[:, 1:2], e3[:, 1:2],
        e1[:, 2:3], e2[:, 2:3], e3[:, 2:3],
    ], -1)

    mk = maskf[...]
    d = dsc[...]
    dscale = jnp.log1p(jnp.exp(d)) * 0.1
    dcol = jnp.broadcast_to(dscale, (N, 1))
    zpad_t = jnp.zeros((N, TW - 85), jnp.float32)
    zpad_p = jnp.zeros((N, PW - 86), jnp.float32)
    table_out[...] = jnp.concatenate([orr, ar, p[:, 0:12], mk, zpad_t], -1)
    pernode_out[...] = jnp.concatenate([ol, al, r9, ca, mk, dcol, zpad_p], -1)


def _stage3_body(pair, gath, pernode, nbr, ln2s, ln2b, ln3s, ln3b,
                 wgh1, wo1, wlr, wgh2, wo2, aw, bw,
                 g_c, h_c, fm_c, a48_c, b48_c, s48_c, s12_c, repa_c, tileb_c,
                 out):
    f32 = jnp.float32

    def mm(a, b):
        return jnp.dot(a, b, preferred_element_type=f32)

    pr = pair[...].reshape(E, P)
    g = gath[...]
    orr_e = g[:, 0:8]
    ar_e = g[:, 8:72]
    pos_e = g[:, 72:84]
    mk_src = g[:, 84:85]

    pn_node = pernode[...]
    per_e = jnp.broadcast_to(pn_node[:, None, :], (BN, K, PW)).reshape(E, PW)
    ol_e = per_e[:, 0:8]
    al_e = per_e[:, 8:72]
    r_e = per_e[:, 72:81]
    t_e = per_e[:, 81:84]
    mk_dst = per_e[:, 84:85]
    dsc_e = per_e[:, 85:86]

    valid = (nbr[...] != -1).astype(f32)
    pmask = mk_dst * mk_src * valid

    def _ln(x, s, b):
        m = jnp.mean(x, -1, keepdims=True)
        v = jnp.mean((x - m) ** 2, -1, keepdims=True)
        return (x - m) * lax.rsqrt(v + 1e-5) * s + b

    pn = _ln(pr, ln2s[...], ln2b[...])
    outer = mm(ol_e, repa_c[...]) * mm(orr_e, tileb_c[...])
    additive = al_e + ar_e
    w1 = wgh1[...]
    gh1 = mm(pn, w1[0:64]) + mm(outer, w1[64:128]) + mm(additive, w1[128:192])
    hidden1 = jax.nn.gelu(gh1[:, 0:128]) * gh1[:, 128:256]
    pair2 = pr + mm(hidden1, wo1[...])

    pn2 = _ln(pair2, ln3s[...], ln3b[...])
    t12 = jnp.concatenate([t_e, t_e, t_e, t_e], -1)
    pm = pos_e - t12
    lp = mm(mm(pm, g_c[...]) * mm(r_e, h_c[...]), fm_c[...])
    dirf = lp * lax.rsqrt(mm(lp * lp, s12_c[...]) + 1e-8)
    wl = wlr[...]
    lr = (mm(pn2, wl[0:64]) + dsc_e * mm(lp, wl[64:76])
          + mm(dirf, wl[76:88]))
    left = lr[:, 0:64]
    right3f = lr[:, 64:128].reshape(BN, K, P)[:, 0:J].reshape(BN * J, P)
    lp3f = lp.reshape(BN, K, 12)[:, 0:J].reshape(BN * J, 12)
    pm3 = pmask.reshape(BN, K)[:, 0:J]

    w2 = wgh2[...]
    wpp = w2[0:64]
    base = mm(left, wpp) + dsc_e * mm(lp, aw[...])
    dscn = jnp.broadcast_to(pn_node[:, 85:86][:, None, :],
                            (BN, J, 1)).reshape(BN * J, 1)
    node_terms = (mm(right3f, wpp) - dscn * mm(lp3f, bw[...]))
    lpB = mm(lp3f, b48_c[...])
    lpA = mm(lp, a48_c[...])
    nt3 = node_terms.reshape(BN, J, 128)
    lpB3 = lpB.reshape(BN, J, 48)

    acc_h = jnp.zeros((E, P), f32)
    cnt = jnp.zeros((E, 1), f32)
    for j in range(J):
        ntj = jnp.broadcast_to(nt3[:, j][:, None, :], (BN, K, 128)).reshape(E, 128)
        lpBj = jnp.broadcast_to(lpB3[:, j][:, None, :], (BN, K, 48)).reshape(E, 48)
        pmj = jnp.broadcast_to(pm3[:, j][:, None, None], (BN, K, 1)).reshape(E, 1)
        rel = lpA - lpBj
        dirs = rel * lax.rsqrt(mm(rel * rel, s48_c[...]) + 1e-8)
        gh2 = base + ntj + mm(dirs, w2[112:160])
        ppm = pmask * pmj
        sel = (ppm > 0).astype(f32)
        acc_h = acc_h + jax.nn.gelu(gh2[:, 0:64]) * gh2[:, 64:128] * sel
        cnt = cnt + ppm
    rec = 1.0 / jnp.maximum(cnt, 1.0)
    out[...] = (pair2 + mm(acc_h, wo2[...]) * rec).reshape(BN, K, P)


def _run_stage1(features, pos12, maskf, ln1s, ln1b, wnode, dsc):
    full2 = lambda w: pl.BlockSpec(w, lambda: (0, 0))
    return pl.pallas_call(
        _stage1_body,
        grid=(),
        in_specs=[
            full2((N, D)), full2((N, 12)), full2((N, 1)),
            full2((1, D)), full2((1, D)), full2((D, 144)), full2((1, 1)),
        ],
        out_specs=[full2((N, TW)), full2((N, PW))],
        out_shape=[
            jax.ShapeDtypeStruct((N, TW), jnp.float32),
            jax.ShapeDtypeStruct((N, PW), jnp.float32),
        ],
    )(features, pos12, maskf, ln1s, ln1b, wnode, dsc)


def _run_sc_gather(table, idx_flat):
    info = plsc.get_sparse_core_info()
    nc, ns = info.num_cores, info.num_subcores
    nw = nc * ns
    b = idx_flat.shape[0]
    b_per_w = b // nw
    bc = 256
    nchunk = b_per_w // bc
    mesh = plsc.VectorSubcoreMesh(core_axis_name="c", subcore_axis_name="s")

    @functools.partial(
        pl.kernel, mesh=mesh,
        out_type=jax.ShapeDtypeStruct((b, TW), jnp.float32),
        scratch_types=[
            pltpu.VMEM((bc,), jnp.int32),
            pltpu.VMEM((bc,), jnp.int32),
            pltpu.VMEM((bc, TW), jnp.float32),
            pltpu.VMEM((bc, TW), jnp.float32),
            pltpu.SemaphoreType.DMA,
            pltpu.SemaphoreType.DMA,
            pltpu.SemaphoreType.DMA,
            pltpu.SemaphoreType.DMA,
            pltpu.SemaphoreType.DMA,
            pltpu.SemaphoreType.DMA,
        ],
    )
    def gather_k(table_hbm, idx_hbm, out_hbm,
                 idx_v0, idx_v1, rows_v0, rows_v1,
                 isem0, isem1, gsem0, gsem1, osem0, osem1):
        wid = lax.axis_index("s") * nc + lax.axis_index("c")
        idx_v = [idx_v0, idx_v1]
        rows_v = [rows_v0, rows_v1]
        isem = [isem0, isem1]
        gsem = [gsem0, gsem1]
        osem = [osem0, osem1]
        h_out = [None, None]
        prev = None
        for c in range(nchunk):
            s = c & 1
            base = wid * b_per_w + c * bc
            if prev is not None:
                pg, ps, pbase = prev
                pg.wait()
                h_out[ps] = pltpu.async_copy(
                    rows_v[ps], out_hbm.at[pl.ds(pbase, bc)], osem[ps])
            if h_out[s] is not None:
                h_out[s].wait()
                h_out[s] = None
            pltpu.async_copy(idx_hbm.at[pl.ds(base, bc)], idx_v[s], isem[s]).wait()
            prev = (pltpu.async_copy(table_hbm.at[idx_v[s]], rows_v[s], gsem[s]),
                    s, base)
        pg, ps, pbase = prev
        pg.wait()
        h_out[ps] = pltpu.async_copy(
            rows_v[ps], out_hbm.at[pl.ds(pbase, bc)], osem[ps])
        for s in range(2):
            if h_out[s] is not None:
                h_out[s].wait()

    return gather_k(table, idx_flat)


def _run_stage3(pair, gathered, pernode, nbr, ln2s, ln2b, ln3s, ln3b,
                wgh1, wo1, wlr, wgh2, wo2, aw, bw, consts):
    n_nodes = pair.shape[0]
    grid = (n_nodes // BN,)
    blk = lambda w: pl.BlockSpec(w, lambda i: (0, 0))
    specs = [
        pl.BlockSpec((BN, K, P), lambda i: (i, 0, 0)),
        pl.BlockSpec((E, TW), lambda i: (i, 0)),
        pl.BlockSpec((BN, PW), lambda i: (i, 0)),
        pl.BlockSpec((E, 1), lambda i: (i, 0)),
        blk((1, P)), blk((1, P)), blk((1, P)), blk((1, P)),
        blk((192, 256)), blk((128, P)), blk((88, 128)), blk((160, 128)), blk((P, P)),
        blk((12, 128)), blk((12, 128)),
        blk((12, 36)), blk((9, 36)), blk((36, 12)),
        blk((12, 48)), blk((12, 48)), blk((48, 48)), blk((12, 12)),
        blk((8, 64)), blk((8, 64)),
    ]
    return pl.pallas_call(
        _stage3_body,
        grid=grid,
        in_specs=specs,
        out_specs=pl.BlockSpec((BN, K, P), lambda i: (i, 0, 0)),
        out_shape=jax.ShapeDtypeStruct((n_nodes, K, P), jnp.float32),
    )(pair, gathered, pernode, nbr.reshape(n_nodes * K, 1), ln2s, ln2b, ln3s, ln3b,
      wgh1, wo1, wlr, wgh2, wo2, aw, bw, *consts)


def kernel(features, pair, pos, neighbours, resi, chain, batch, mask,
           ln1_s, ln1_b, ln2_s, ln2_b, ln3_s, ln3_b,
           W_ol, W_or, W_al, W_ar, W_g1, W_h1, W_o1,
           W_l, W_r, W_g2, W_h2, W_o2, d_scale):
    f32 = jnp.float32
    maskf = mask.astype(f32).reshape(N, 1)
    pos12 = pos.astype(f32).reshape(N, 12)
    nbr = neighbours.astype(jnp.int32)
    dsc = jnp.asarray(d_scale, f32).reshape(1, 1)
    consts = tuple(jnp.asarray(c) for c in _CONSTS)
    a48, b48 = consts[3], consts[4]
    wnode = jnp.concatenate([W_ol, W_or, W_al, W_ar], axis=1)
    wgh1 = jnp.concatenate([W_g1, W_h1], axis=1)
    wlr = jnp.concatenate([W_l, W_r], axis=1)
    wgh2 = jnp.concatenate([W_g2, W_h2], axis=1)
    aw = jnp.dot(a48, wgh2[64:112])
    bw = jnp.dot(b48, wgh2[64:112])

    table, pernode = _run_stage1(
        features.astype(f32), pos12, maskf,
        ln1_s.reshape(1, D), ln1_b.reshape(1, D), wnode, dsc)

    gathered = _run_sc_gather(table, nbr.reshape(N * K))
    return _run_stage3(
        pair.astype(f32), gathered, pernode, nbr,
        ln2_s.reshape(1, P), ln2_b.reshape(1, P),
        ln3_s.reshape(1, P), ln3_b.reshape(1, P),
        wgh1, W_o1, wlr, wgh2, W_o2, aw, bw, consts)
